# baseline (speedup 1.0000x reference)
import jax
import jax.numpy as jnp
from jax import lax
from jax.experimental import pallas as pl
from jax.experimental.pallas import tpu as pltpu
from jax.experimental.pallas import tpu_sc as plsc

_B, _N, _C = 4, 8192, 1024
_NC, _NS, _L = 2, 16, 16
_NW = _NC * _NS
_ROWS_PER_W = _N // _NW
_R = 32
_RE = 16
_RB = _ROWS_PER_W // _R
_NBUF = 2
_UNROLL = 8
_STEPS = _RB * _B


def _sc_body(tok_hbm, emb_hbm, out_hbm, emb0, emb1,
             tok0, tok1,
             esem0, esem1, isem0, isem1, osem0, osem1):
    emb_bufs = (emb0, emb1)
    emb_sems = (esem0, esem1)
    tok_bufs = (tok0, tok1)
    in_sems = (isem0, isem1)
    out_sems = (osem0, osem1)
    wid = lax.axis_index("s") * _NC + lax.axis_index("c")
    row_base = wid * _ROWS_PER_W

    def rows(step):
        rb, b = step // _B, step % _B
        return b, row_base + rb * _R

    def start_in(step):
        p = step % _NBUF
        b, r0 = rows(step)
        pltpu.async_copy(
            tok_hbm.at[b, pl.ds(r0, _R), :], tok_bufs[p], in_sems[p])

    def wait_in(step):
        p = step % _NBUF
        b, r0 = rows(step)
        pltpu.make_async_copy(
            tok_hbm.at[b, pl.ds(r0, _R), :], tok_bufs[p], in_sems[p]).wait()

    def start_out(step):
        p = step % _NBUF
        b, r0 = rows(step)
        pltpu.async_copy(
            tok_bufs[p], out_hbm.at[b, pl.ds(r0, _R), :], out_sems[p])

    def wait_out(step):
        p = step % _NBUF
        b, r0 = rows(step)
        pltpu.make_async_copy(
            tok_bufs[p], out_hbm.at[b, pl.ds(r0, _R), :], out_sems[p]).wait()

    def start_emb(rb):
        r0 = row_base + rb * _R
        for h in range(2):
            pltpu.async_copy(
                emb_hbm.at[pl.ds(r0 + h * _RE, _RE), :],
                emb_bufs[h], emb_sems[h])

    def wait_emb(rb):
        r0 = row_base + rb * _R
        for h in range(2):
            pltpu.make_async_copy(
                emb_hbm.at[pl.ds(r0 + h * _RE, _RE), :],
                emb_bufs[h], emb_sems[h]).wait()

    start_emb(0)
    for s in range(_NBUF - 1):
        start_in(s)

    for s in range(_STEPS):
        p = s % _NBUF
        rb, b = s // _B, s % _B
        if b == 0:
            wait_emb(rb)
        wait_in(s)

        for h in range(2):
            emb_v = emb_bufs[h]

            @plsc.parallel_loop(0, _RE * _C, _L, unroll=_UNROLL)
            def _add(i):
                r = i >> 10
                c = pl.multiple_of(i & (_C - 1), _L)
                plsc.addupdate(tok_bufs[p].at[r + h * _RE, pl.ds(c, _L)],
                               emb_v[r, pl.ds(c, _L)])

        if b == _B - 1 and rb + 1 < _RB:
            start_emb(rb + 1)
        if s >= 1:
            wait_out(s - 1)
        if s + _NBUF - 1 < _STEPS:
            start_in(s + _NBUF - 1)
        start_out(s)

    wait_out(_STEPS - 1)


@jax.jit
def _sc_add(tokens, emb):
    mesh = plsc.VectorSubcoreMesh(core_axis_name="c", subcore_axis_name="s")
    return pl.kernel(
        _sc_body,
        out_type=jax.ShapeDtypeStruct((_B, _N, _C), jnp.float32),
        mesh=mesh,
        scratch_types=(
            [pltpu.VMEM((_RE, _C), jnp.float32) for _ in range(2)]
            + [pltpu.VMEM((_R, _C), jnp.float32) for _ in range(_NBUF)]
            + [pltpu.SemaphoreType.DMA for _ in range(2 + 2 * _NBUF)]
        ),
    )(tokens, emb)


def kernel(tokens, emb):
    return _sc_add(tokens, emb)


# speedup vs baseline: 1.4599x; 1.4599x over previous
import jax
import jax.numpy as jnp
from jax import lax
from jax.experimental import pallas as pl
from jax.experimental.pallas import tpu as pltpu
from jax.experimental.pallas import tpu_sc as plsc

_B, _N, _C = 4, 8192, 1024
_NC, _NS, _L = 2, 16, 16
_NW = _NC * _NS
_ROWS_PER_W = _N // _NW
_R = 16
_RB = _ROWS_PER_W // _R
_NBUF = 4
_UNROLL = 8
_STEPS = _RB * _B


def _sc_body(tok_hbm, emb_hbm, out_hbm, emb0, emb1,
             tok0, tok1, tok2, tok3,
             esem0, esem1, isem0, isem1, isem2, isem3,
             osem0, osem1, osem2, osem3):
    emb_bufs = (emb0, emb1)
    emb_sems = (esem0, esem1)
    tok_bufs = (tok0, tok1, tok2, tok3)
    in_sems = (isem0, isem1, isem2, isem3)
    out_sems = (osem0, osem1, osem2, osem3)
    wid = lax.axis_index("s") * _NC + lax.axis_index("c")
    row_base = wid * _ROWS_PER_W

    def rows(step):
        rb, b = step // _B, step % _B
        return b, row_base + rb * _R

    def start_in(step):
        p = step % _NBUF
        b, r0 = rows(step)
        pltpu.async_copy(
            tok_hbm.at[b, pl.ds(r0, _R), :], tok_bufs[p], in_sems[p])

    def wait_in(step):
        p = step % _NBUF
        b, r0 = rows(step)
        pltpu.make_async_copy(
            tok_hbm.at[b, pl.ds(r0, _R), :], tok_bufs[p], in_sems[p]).wait()

    def start_out(step):
        p = step % _NBUF
        b, r0 = rows(step)
        pltpu.async_copy(
            tok_bufs[p], out_hbm.at[b, pl.ds(r0, _R), :], out_sems[p])

    def wait_out(step):
        p = step % _NBUF
        b, r0 = rows(step)
        pltpu.make_async_copy(
            tok_bufs[p], out_hbm.at[b, pl.ds(r0, _R), :], out_sems[p]).wait()

    def start_emb(rb):
        pltpu.async_copy(
            emb_hbm.at[pl.ds(row_base + rb * _R, _R), :],
            emb_bufs[rb % 2], emb_sems[rb % 2])

    def wait_emb(rb):
        pltpu.make_async_copy(
            emb_hbm.at[pl.ds(row_base + rb * _R, _R), :],
            emb_bufs[rb % 2], emb_sems[rb % 2]).wait()

    start_emb(0)
    for s in range(_NBUF - 1):
        start_in(s)

    for s in range(_STEPS):
        p = s % _NBUF
        rb, b = s // _B, s % _B
        if b == 0:
            wait_emb(rb)
        if b == 1 and rb + 1 < _RB:
            start_emb(rb + 1)
        wait_in(s)
        emb_v = emb_bufs[rb % 2]

        @plsc.parallel_loop(0, _R * _C, _L, unroll=_UNROLL)
        def _add(i):
            r = i >> 10
            c = pl.multiple_of(i & (_C - 1), _L)
            plsc.addupdate(tok_bufs[p].at[r, pl.ds(c, _L)],
                           emb_v[r, pl.ds(c, _L)])

        if s >= 1:
            wait_out(s - 1)
        if s + _NBUF - 1 < _STEPS:
            start_in(s + _NBUF - 1)
        start_out(s)

    wait_out(_STEPS - 1)


@jax.jit
def _sc_add(tokens, emb):
    mesh = plsc.VectorSubcoreMesh(core_axis_name="c", subcore_axis_name="s")
    return pl.kernel(
        _sc_body,
        out_type=jax.ShapeDtypeStruct((_B, _N, _C), jnp.float32),
        mesh=mesh,
        scratch_types=(
            [pltpu.VMEM((_R, _C), jnp.float32) for _ in range(2)]
            + [pltpu.VMEM((_R, _C), jnp.float32) for _ in range(_NBUF)]
            + [pltpu.SemaphoreType.DMA for _ in range(2 + 2 * _NBUF)]
        ),
    )(tokens, emb)


def kernel(tokens, emb):
    return _sc_add(tokens, emb)


# speedup vs baseline: 1.4794x; 1.0134x over previous
import jax
import jax.numpy as jnp
from jax import lax
from jax.experimental import pallas as pl
from jax.experimental.pallas import tpu as pltpu
from jax.experimental.pallas import tpu_sc as plsc

_B, _N, _C = 4, 8192, 1024
_NC, _NS, _L = 2, 16, 16
_NW = _NC * _NS
_ROWS_PER_W = _N // _NW
_R = 16
_RB = _ROWS_PER_W // _R
_NBUF = 4
_UNROLL = 8
_STEPS = _RB * _B


def _sc_body(tok_hbm, emb_hbm, out_hbm, emb0, emb1,
             tok0, tok1, tok2, tok3,
             esem0, esem1, isem0, isem1, isem2, isem3,
             osem0, osem1, osem2, osem3):
    emb_bufs = (emb0, emb1)
    emb_sems = (esem0, esem1)
    tok_bufs = (tok0, tok1, tok2, tok3)
    in_sems = (isem0, isem1, isem2, isem3)
    out_sems = (osem0, osem1, osem2, osem3)
    wid = lax.axis_index("s") * _NC + lax.axis_index("c")
    row_base = wid * _ROWS_PER_W

    def rows(step):
        rb, b = step // _B, step % _B
        return b, (rb * _NW + wid) * _R

    def start_in(step):
        p = step % _NBUF
        b, r0 = rows(step)
        pltpu.async_copy(
            tok_hbm.at[b, pl.ds(r0, _R), :], tok_bufs[p], in_sems[p])

    def wait_in(step):
        p = step % _NBUF
        b, r0 = rows(step)
        pltpu.make_async_copy(
            tok_hbm.at[b, pl.ds(r0, _R), :], tok_bufs[p], in_sems[p]).wait()

    def start_out(step):
        p = step % _NBUF
        b, r0 = rows(step)
        pltpu.async_copy(
            tok_bufs[p], out_hbm.at[b, pl.ds(r0, _R), :], out_sems[p])

    def wait_out(step):
        p = step % _NBUF
        b, r0 = rows(step)
        pltpu.make_async_copy(
            tok_bufs[p], out_hbm.at[b, pl.ds(r0, _R), :], out_sems[p]).wait()

    def start_emb(rb):
        pltpu.async_copy(
            emb_hbm.at[pl.ds((rb * _NW + wid) * _R, _R), :],
            emb_bufs[rb % 2], emb_sems[rb % 2])

    def wait_emb(rb):
        pltpu.make_async_copy(
            emb_hbm.at[pl.ds((rb * _NW + wid) * _R, _R), :],
            emb_bufs[rb % 2], emb_sems[rb % 2]).wait()

    start_emb(0)
    for s in range(_NBUF - 1):
        start_in(s)

    for s in range(_STEPS):
        p = s % _NBUF
        rb, b = s // _B, s % _B
        if b == 0:
            wait_emb(rb)
        if b == 1 and rb + 1 < _RB:
            start_emb(rb + 1)
        wait_in(s)
        emb_v = emb_bufs[rb % 2]

        @plsc.parallel_loop(0, _R * _C, _L, unroll=_UNROLL)
        def _add(i):
            r = i >> 10
            c = pl.multiple_of(i & (_C - 1), _L)
            plsc.addupdate(tok_bufs[p].at[r, pl.ds(c, _L)],
                           emb_v[r, pl.ds(c, _L)])

        if s >= 1:
            wait_out(s - 1)
        if s + _NBUF - 1 < _STEPS:
            start_in(s + _NBUF - 1)
        start_out(s)

    wait_out(_STEPS - 1)


@jax.jit
def _sc_add(tokens, emb):
    mesh = plsc.VectorSubcoreMesh(core_axis_name="c", subcore_axis_name="s")
    return pl.kernel(
        _sc_body,
        out_type=jax.ShapeDtypeStruct((_B, _N, _C), jnp.float32),
        mesh=mesh,
        scratch_types=(
            [pltpu.VMEM((_R, _C), jnp.float32) for _ in range(2)]
            + [pltpu.VMEM((_R, _C), jnp.float32) for _ in range(_NBUF)]
            + [pltpu.SemaphoreType.DMA for _ in range(2 + 2 * _NBUF)]
        ),
    )(tokens, emb)


def kernel(tokens, emb):
    return _sc_add(tokens, emb)
